# Initial kernel scaffold; baseline (speedup 1.0000x reference)
import jax
import jax.numpy as jnp
from jax.experimental import pallas as pl


def kernel(x, edge_index, W1, b1, W2, b2):
    raise NotImplementedError("write your pallas kernel here")


# speedup vs baseline: 18.9724x; 18.9724x over previous
import functools

import jax
import jax.numpy as jnp
from jax import lax
from jax.experimental import pallas as pl
from jax.experimental.pallas import tpu as pltpu
from jax.experimental.pallas import tpu_sc as plsc

N = 10000
E = 320000
D1 = 128
D2 = 16
NCLS = 8

NC = 2
NS = 16
NW = NC * NS
EPW = E // NW
CH = 80
NCHUNK = EPW // CH
RPT = 624
TAIL = N - NS * RPT

_mesh = plsc.VectorSubcoreMesh(core_axis_name="c", subcore_axis_name="s")


def _make_agg(D):
  @functools.partial(
      pl.kernel,
      mesh=_mesh,
      compiler_params=pltpu.CompilerParams(use_tc_tiling_on_sc=False),
      out_type=(
          jax.ShapeDtypeStruct((N, D), jnp.float32),
          jax.ShapeDtypeStruct((N, D), jnp.float32),
      ),
      scratch_types=[
          pltpu.VMEM((EPW,), jnp.int32),
          pltpu.VMEM((NCHUNK, CH), jnp.int32),
          pltpu.VMEM((CH, D), jnp.float32),
          pltpu.VMEM_SHARED((N, D), jnp.float32),
      ],
  )
  def agg(z_hbm, src_hbm, dst3_hbm, zero_hbm, outa, outb,
          sidx_v, didx_v, rows_v, acc_sh):
    c = lax.axis_index("c")
    s = lax.axis_index("s")
    wid = s * NC + c
    pltpu.sync_copy(src_hbm.at[pl.ds(pl.multiple_of(wid * EPW, 8), EPW)],
                    sidx_v)
    pltpu.sync_copy(dst3_hbm.at[wid], didx_v)
    r0 = pl.multiple_of(s * RPT, 8)
    pltpu.sync_copy(zero_hbm.at[pl.ds(r0, RPT)], acc_sh.at[pl.ds(r0, RPT)])

    @pl.when(s == 0)
    def _():
      pltpu.sync_copy(zero_hbm.at[pl.ds(NS * RPT, TAIL)],
                      acc_sh.at[pl.ds(NS * RPT, TAIL)])

    plsc.subcore_barrier()

    def body(j, carry):
      off = pl.multiple_of(j * CH, 8)
      pltpu.sync_copy(z_hbm.at[sidx_v.at[pl.ds(off, CH)]], rows_v)
      pltpu.sync_copy(rows_v, acc_sh.at[didx_v.at[j]], add=True)
      return carry

    lax.fori_loop(0, NCHUNK, body, 0)
    plsc.subcore_barrier()

    @pl.when(c == 0)
    def _():
      pltpu.sync_copy(acc_sh.at[pl.ds(r0, RPT)], outa.at[pl.ds(r0, RPT)])

      @pl.when(s == 0)
      def _():
        pltpu.sync_copy(acc_sh.at[pl.ds(NS * RPT, TAIL)],
                        outa.at[pl.ds(NS * RPT, TAIL)])

    @pl.when(c == 1)
    def _():
      pltpu.sync_copy(acc_sh.at[pl.ds(r0, RPT)], outb.at[pl.ds(r0, RPT)])

      @pl.when(s == 0)
      def _():
        pltpu.sync_copy(acc_sh.at[pl.ds(NS * RPT, TAIL)],
                        outb.at[pl.ds(NS * RPT, TAIL)])

  return agg


_agg128 = _make_agg(D1)
_agg16 = _make_agg(D2)

BLK = 1000


def _z1_body(x_ref, w_ref, da_ref, db_ref, o_ref):
  deg = da_ref[:, 0:1] + db_ref[:, 0:1] + 1.0
  dis = lax.rsqrt(deg)
  h = jnp.dot(x_ref[...], w_ref[...], preferred_element_type=jnp.float32)
  o_ref[...] = h * dis


def _z2_body(sa_ref, sb_ref, z1_ref, da_ref, db_ref, b1_ref, w2_ref, o_ref):
  deg = da_ref[:, 0:1] + db_ref[:, 0:1] + 1.0
  dis = lax.rsqrt(deg)
  h = (sa_ref[...] + sb_ref[...] + z1_ref[...]) * dis + b1_ref[...]
  h = jnp.maximum(h, 0.0)
  o_ref[...] = jnp.dot(h, w2_ref[...], preferred_element_type=jnp.float32) * dis


def _out_body(sa_ref, sb_ref, z2_ref, da_ref, db_ref, b2_ref, o_ref):
  deg = da_ref[:, 0:1] + db_ref[:, 0:1] + 1.0
  dis = lax.rsqrt(deg)
  y = (sa_ref[...] + sb_ref[...] + z2_ref[...]) * dis
  o_ref[...] = y[:, 0:NCLS] + b2_ref[...]


def _row_spec(d):
  return pl.BlockSpec((BLK, d), lambda i: (i, 0))


def _full_spec(r, c):
  return pl.BlockSpec((r, c), lambda i: (0, 0))


def kernel(x, edge_index, W1, b1, W2, b2):
  src = edge_index[0].astype(jnp.int32)
  dst = edge_index[1].astype(jnp.int32)
  dst3 = dst.reshape(NW, NCHUNK, CH)
  ones16 = jnp.ones((N, D2), jnp.float32)
  zeros16 = jnp.zeros((N, D2), jnp.float32)
  zeros128 = jnp.zeros((N, D1), jnp.float32)
  W2p = jnp.pad(W2, ((0, 0), (0, D2 - NCLS)))
  b1r = b1.reshape(1, D1)
  b2r = b2.reshape(1, NCLS)

  dega, degb = _agg16(ones16, src, dst3, zeros16)

  z1 = pl.pallas_call(
      _z1_body,
      grid=(N // BLK,),
      in_specs=[_row_spec(D1), _full_spec(D1, D1), _row_spec(D2),
                _row_spec(D2)],
      out_specs=_row_spec(D1),
      out_shape=jax.ShapeDtypeStruct((N, D1), jnp.float32),
  )(x, W1, dega, degb)

  s1a, s1b = _agg128(z1, src, dst3, zeros128)

  z2 = pl.pallas_call(
      _z2_body,
      grid=(N // BLK,),
      in_specs=[_row_spec(D1), _row_spec(D1), _row_spec(D1), _row_spec(D2),
                _row_spec(D2), _full_spec(1, D1), _full_spec(D1, D2)],
      out_specs=_row_spec(D2),
      out_shape=jax.ShapeDtypeStruct((N, D2), jnp.float32),
  )(s1a, s1b, z1, dega, degb, b1r, W2p)

  s2a, s2b = _agg16(z2, src, dst3, zeros16)

  out = pl.pallas_call(
      _out_body,
      grid=(N // BLK,),
      in_specs=[_row_spec(D2), _row_spec(D2), _row_spec(D2), _row_spec(D2),
                _row_spec(D2), _full_spec(1, NCLS)],
      out_specs=_row_spec(NCLS),
      out_shape=jax.ShapeDtypeStruct((N, NCLS), jnp.float32),
  )(s2a, s2b, z2, dega, degb, b2r)

  return out


# speedup vs baseline: 28.9748x; 1.5272x over previous
import functools

import jax
import jax.numpy as jnp
from jax import lax
from jax.experimental import pallas as pl
from jax.experimental.pallas import tpu as pltpu
from jax.experimental.pallas import tpu_sc as plsc

N = 10000
E = 320000
D1 = 128
DH = 64
D2 = 16
NCLS = 8

NC = 2
NS = 16
NW = NC * NS
EPW = E // NW
CH = 80
NCHUNK = EPW // CH
K = 5
BATCH = K * CH
NBATCH = EPW // BATCH
RPT = 624
TAIL = N - NS * RPT

_mesh = plsc.VectorSubcoreMesh(core_axis_name="c", subcore_axis_name="s")
_sc_params = pltpu.CompilerParams(use_tc_tiling_on_sc=False)


def _worker_prologue(dst3_hbm, didx_v):
  c = lax.axis_index("c")
  s = lax.axis_index("s")
  wid = s * NC + c
  pltpu.sync_copy(dst3_hbm.at[wid], didx_v)
  return c, s, wid


def _zero_acc(zero_hbm, acc_sh, s):
  r0 = pl.multiple_of(s * RPT, 8)
  pltpu.sync_copy(zero_hbm.at[pl.ds(r0, RPT)], acc_sh.at[pl.ds(r0, RPT)])

  @pl.when(s == 0)
  def _():
    pltpu.sync_copy(zero_hbm.at[pl.ds(NS * RPT, TAIL)],
                    acc_sh.at[pl.ds(NS * RPT, TAIL)])


def _copy_out(acc_sh, outa, outb, c, s):
  r0 = pl.multiple_of(s * RPT, 8)

  @pl.when(c == 0)
  def _():
    pltpu.sync_copy(acc_sh.at[pl.ds(r0, RPT)], outa.at[pl.ds(r0, RPT)])

    @pl.when(s == 0)
    def _():
      pltpu.sync_copy(acc_sh.at[pl.ds(NS * RPT, TAIL)],
                      outa.at[pl.ds(NS * RPT, TAIL)])

  @pl.when(c == 1)
  def _():
    pltpu.sync_copy(acc_sh.at[pl.ds(r0, RPT)], outb.at[pl.ds(r0, RPT)])

    @pl.when(s == 0)
    def _():
      pltpu.sync_copy(acc_sh.at[pl.ds(NS * RPT, TAIL)],
                      outb.at[pl.ds(NS * RPT, TAIL)])


def _run_edge_pass(z_hbm, sidx_v, didx_v, buf, acc_sh, sem_g, sem_s):
  def body(t, carry):
    gathers = []
    for b in range(K):
      off = pl.multiple_of((t * K + b) * CH, 8)
      gathers.append(
          pltpu.async_copy(z_hbm.at[sidx_v.at[pl.ds(off, CH)]],
                           buf.at[pl.ds(b * CH, CH)], sem_g))
    for h in gathers:
      h.wait()
    scatters = []
    for b in range(K):
      scatters.append(
          pltpu.async_copy(buf.at[pl.ds(b * CH, CH)],
                           acc_sh.at[didx_v.at[t * K + b]], sem_s, add=True))
    for h in scatters:
      h.wait()
    return carry

  lax.fori_loop(0, NBATCH, body, 0)


@functools.partial(
    pl.kernel,
    mesh=_mesh,
    compiler_params=_sc_params,
    out_type=tuple(jax.ShapeDtypeStruct((N, DH), jnp.float32)
                   for _ in range(4)),
    scratch_types=[
        pltpu.VMEM((EPW,), jnp.int32),
        pltpu.VMEM((NCHUNK, CH), jnp.int32),
        pltpu.VMEM((BATCH, DH), jnp.float32),
        pltpu.VMEM_SHARED((N, DH), jnp.float32),
        pltpu.SemaphoreType.DMA,
        pltpu.SemaphoreType.DMA,
    ],
)
def _agg64x2(za_hbm, zb_hbm, src_hbm, dst3_hbm, zero_hbm,
             oa0, ob0, oa1, ob1,
             sidx_v, didx_v, buf, acc_sh, sem_g, sem_s):
  c, s, wid = _worker_prologue(dst3_hbm, didx_v)
  pltpu.sync_copy(src_hbm.at[pl.ds(pl.multiple_of(wid * EPW, 8), EPW)],
                  sidx_v)
  _zero_acc(zero_hbm, acc_sh, s)
  plsc.subcore_barrier()

  _run_edge_pass(za_hbm, sidx_v, didx_v, buf, acc_sh, sem_g, sem_s)
  plsc.subcore_barrier()
  _copy_out(acc_sh, oa0, ob0, c, s)
  _zero_acc(zero_hbm, acc_sh, s)
  plsc.subcore_barrier()

  _run_edge_pass(zb_hbm, sidx_v, didx_v, buf, acc_sh, sem_g, sem_s)
  plsc.subcore_barrier()
  _copy_out(acc_sh, oa1, ob1, c, s)


@functools.partial(
    pl.kernel,
    mesh=_mesh,
    compiler_params=_sc_params,
    out_type=(
        jax.ShapeDtypeStruct((N, D2), jnp.float32),
        jax.ShapeDtypeStruct((N, D2), jnp.float32),
    ),
    scratch_types=[
        pltpu.VMEM((EPW,), jnp.int32),
        pltpu.VMEM((NCHUNK, CH), jnp.int32),
        pltpu.VMEM((BATCH, D2), jnp.float32),
        pltpu.VMEM_SHARED((N, D2), jnp.float32),
        pltpu.SemaphoreType.DMA,
        pltpu.SemaphoreType.DMA,
    ],
)
def _agg16(z_hbm, src_hbm, dst3_hbm, zero_hbm, outa, outb,
           sidx_v, didx_v, buf, acc_sh, sem_g, sem_s):
  c, s, wid = _worker_prologue(dst3_hbm, didx_v)
  pltpu.sync_copy(src_hbm.at[pl.ds(pl.multiple_of(wid * EPW, 8), EPW)],
                  sidx_v)
  _zero_acc(zero_hbm, acc_sh, s)
  plsc.subcore_barrier()
  _run_edge_pass(z_hbm, sidx_v, didx_v, buf, acc_sh, sem_g, sem_s)
  plsc.subcore_barrier()
  _copy_out(acc_sh, outa, outb, c, s)


@functools.partial(
    pl.kernel,
    mesh=_mesh,
    compiler_params=_sc_params,
    out_type=(
        jax.ShapeDtypeStruct((N, D2), jnp.float32),
        jax.ShapeDtypeStruct((N, D2), jnp.float32),
    ),
    scratch_types=[
        pltpu.VMEM((NCHUNK, CH), jnp.int32),
        pltpu.VMEM((CH, D2), jnp.float32),
        pltpu.VMEM_SHARED((N, D2), jnp.float32),
        pltpu.SemaphoreType.DMA,
    ],
)
def _deg_kernel(dst3_hbm, ones_hbm, zero_hbm, outa, outb,
                didx_v, ones_v, acc_sh, sem):
  c, s, wid = _worker_prologue(dst3_hbm, didx_v)
  pltpu.sync_copy(ones_hbm, ones_v)
  _zero_acc(zero_hbm, acc_sh, s)
  plsc.subcore_barrier()

  def body(t, carry):
    handles = []
    for b in range(K):
      handles.append(
          pltpu.async_copy(ones_v, acc_sh.at[didx_v.at[t * K + b]], sem,
                           add=True))
    for h in handles:
      h.wait()
    return carry

  lax.fori_loop(0, NBATCH, body, 0)
  plsc.subcore_barrier()
  _copy_out(acc_sh, outa, outb, c, s)


BLK = 1000


def _p1_body(x_ref, w_ref, o_ref):
  o_ref[...] = jnp.dot(x_ref[...], w_ref[...],
                       preferred_element_type=jnp.float32)


def _z1_body(p_ref, da_ref, db_ref, oa_ref, ob_ref):
  deg = da_ref[:, 0:1] + db_ref[:, 0:1] + 1.0
  dis = lax.rsqrt(deg)
  z = p_ref[...] * dis
  oa_ref[...] = z[:, 0:DH]
  ob_ref[...] = z[:, DH:D1]


def _z2_body(s0a_ref, s0b_ref, s1a_ref, s1b_ref, za_ref, zb_ref,
             da_ref, db_ref, b1_ref, w2_ref, o_ref):
  deg = da_ref[:, 0:1] + db_ref[:, 0:1] + 1.0
  dis = lax.rsqrt(deg)
  hl = (s0a_ref[...] + s0b_ref[...] + za_ref[...]) * dis + b1_ref[:, 0:DH]
  hh = (s1a_ref[...] + s1b_ref[...] + zb_ref[...]) * dis + b1_ref[:, DH:D1]
  h = jnp.maximum(jnp.concatenate([hl, hh], axis=1), 0.0)
  o_ref[...] = jnp.dot(h, w2_ref[...], preferred_element_type=jnp.float32) * dis


def _out_body(sa_ref, sb_ref, z2_ref, da_ref, db_ref, b2_ref, o_ref):
  deg = da_ref[:, 0:1] + db_ref[:, 0:1] + 1.0
  dis = lax.rsqrt(deg)
  y = (sa_ref[...] + sb_ref[...] + z2_ref[...]) * dis
  o_ref[...] = y[:, 0:NCLS] + b2_ref[...]


def _row_spec(d):
  return pl.BlockSpec((BLK, d), lambda i: (i, 0))


def _full_spec(r, c):
  return pl.BlockSpec((r, c), lambda i: (0, 0))


def kernel(x, edge_index, W1, b1, W2, b2):
  src = edge_index[0].astype(jnp.int32)
  dst = edge_index[1].astype(jnp.int32)
  dst3 = dst.reshape(NW, NCHUNK, CH)
  ones_rows = jnp.ones((CH, D2), jnp.float32)
  zeros16 = jnp.zeros((N, D2), jnp.float32)
  zeros64 = jnp.zeros((N, DH), jnp.float32)
  W2p = jnp.pad(W2, ((0, 0), (0, D2 - NCLS)))
  b1r = b1.reshape(1, D1)
  b2r = b2.reshape(1, NCLS)

  dega, degb = _deg_kernel(dst3, ones_rows, zeros16)

  p1 = pl.pallas_call(
      _p1_body,
      grid=(N // BLK,),
      in_specs=[_row_spec(D1), _full_spec(D1, D1)],
      out_specs=_row_spec(D1),
      out_shape=jax.ShapeDtypeStruct((N, D1), jnp.float32),
  )(x, W1)

  z1a, z1b = pl.pallas_call(
      _z1_body,
      grid=(N // BLK,),
      in_specs=[_row_spec(D1), _row_spec(D2), _row_spec(D2)],
      out_specs=(_row_spec(DH), _row_spec(DH)),
      out_shape=(jax.ShapeDtypeStruct((N, DH), jnp.float32),
                 jax.ShapeDtypeStruct((N, DH), jnp.float32)),
  )(p1, dega, degb)

  s0a, s0b, s1a, s1b = _agg64x2(z1a, z1b, src, dst3, zeros64)

  z2 = pl.pallas_call(
      _z2_body,
      grid=(N // BLK,),
      in_specs=[_row_spec(DH)] * 6 + [_row_spec(D2), _row_spec(D2),
                _full_spec(1, D1), _full_spec(D1, D2)],
      out_specs=_row_spec(D2),
      out_shape=jax.ShapeDtypeStruct((N, D2), jnp.float32),
  )(s0a, s0b, s1a, s1b, z1a, z1b, dega, degb, b1r, W2p)

  s2a, s2b = _agg16(z2, src, dst3, zeros16)

  out = pl.pallas_call(
      _out_body,
      grid=(N // BLK,),
      in_specs=[_row_spec(D2), _row_spec(D2), _row_spec(D2), _row_spec(D2),
                _row_spec(D2), _full_spec(1, NCLS)],
      out_specs=_row_spec(NCLS),
      out_shape=jax.ShapeDtypeStruct((N, NCLS), jnp.float32),
  )(s2a, s2b, z2, dega, degb, b2r)

  return out


# speedup vs baseline: 30.3968x; 1.0491x over previous
import functools

import jax
import jax.numpy as jnp
from jax import lax
from jax.experimental import pallas as pl
from jax.experimental.pallas import tpu as pltpu
from jax.experimental.pallas import tpu_sc as plsc

N = 10000
E = 320000
D1 = 128
DH = 64
D2 = 16
NCLS = 8

NC = 2
NS = 16
NW = NC * NS
EPW = E // NW
CH = 80
NCHUNK = EPW // CH
K = 5
BATCH = K * CH
NBATCH = EPW // BATCH
RPT = 624
TAIL = N - NS * RPT

_mesh = plsc.VectorSubcoreMesh(core_axis_name="c", subcore_axis_name="s")
_sc_params = pltpu.CompilerParams(use_tc_tiling_on_sc=False)


def _worker_prologue(dst3_hbm, didx_v):
  c = lax.axis_index("c")
  s = lax.axis_index("s")
  wid = s * NC + c
  pltpu.sync_copy(dst3_hbm.at[wid], didx_v)
  return c, s, wid


def _zero_acc(zero_hbm, acc_sh, s):
  r0 = pl.multiple_of(s * RPT, 8)
  pltpu.sync_copy(zero_hbm.at[pl.ds(r0, RPT)], acc_sh.at[pl.ds(r0, RPT)])

  @pl.when(s == 0)
  def _():
    pltpu.sync_copy(zero_hbm.at[pl.ds(NS * RPT, TAIL)],
                    acc_sh.at[pl.ds(NS * RPT, TAIL)])


def _copy_out(acc_sh, outa, outb, c, s):
  r0 = pl.multiple_of(s * RPT, 8)

  @pl.when(c == 0)
  def _():
    pltpu.sync_copy(acc_sh.at[pl.ds(r0, RPT)], outa.at[pl.ds(r0, RPT)])

    @pl.when(s == 0)
    def _():
      pltpu.sync_copy(acc_sh.at[pl.ds(NS * RPT, TAIL)],
                      outa.at[pl.ds(NS * RPT, TAIL)])

  @pl.when(c == 1)
  def _():
    pltpu.sync_copy(acc_sh.at[pl.ds(r0, RPT)], outb.at[pl.ds(r0, RPT)])

    @pl.when(s == 0)
    def _():
      pltpu.sync_copy(acc_sh.at[pl.ds(NS * RPT, TAIL)],
                      outb.at[pl.ds(NS * RPT, TAIL)])


def _run_edge_pass(z_hbm, sidx_v, didx_v, buf, acc_sh, sem_g, sem_s,
                   nbatch=NBATCH):
  def body(t, carry):
    gathers = []
    for b in range(K):
      off = pl.multiple_of((t * K + b) * CH, 8)
      gathers.append(
          pltpu.async_copy(z_hbm.at[sidx_v.at[pl.ds(off, CH)]],
                           buf.at[pl.ds(b * CH, CH)], sem_g))
    for h in gathers:
      h.wait()
    scatters = []
    for b in range(K):
      scatters.append(
          pltpu.async_copy(buf.at[pl.ds(b * CH, CH)],
                           acc_sh.at[didx_v.at[t * K + b]], sem_s, add=True))
    for h in scatters:
      h.wait()
    return carry

  lax.fori_loop(0, nbatch, body, 0)


@functools.partial(
    pl.kernel,
    mesh=_mesh,
    compiler_params=_sc_params,
    out_type=(
        jax.ShapeDtypeStruct((N, DH), jnp.float32),
        jax.ShapeDtypeStruct((N, DH), jnp.float32),
    ),
    scratch_types=[
        pltpu.VMEM((E // NS,), jnp.int32),
        pltpu.VMEM((E // NS // CH, CH), jnp.int32),
        pltpu.VMEM((BATCH, DH), jnp.float32),
        pltpu.VMEM_SHARED((N, DH), jnp.float32),
        pltpu.SemaphoreType.DMA,
        pltpu.SemaphoreType.DMA,
    ],
)
def _agg64split(za_hbm, zb_hbm, src_hbm, dst3_hbm, zero_hbm, o0, o1,
                sidx_v, didx_v, buf, acc_sh, sem_g, sem_s):
  c = lax.axis_index("c")
  s = lax.axis_index("s")
  epw = E // NS
  pltpu.sync_copy(dst3_hbm.at[s], didx_v)
  pltpu.sync_copy(src_hbm.at[pl.ds(pl.multiple_of(s * epw, 8), epw)],
                  sidx_v)
  _zero_acc(zero_hbm, acc_sh, s)
  plsc.subcore_barrier()

  @pl.when(c == 0)
  def _():
    _run_edge_pass(za_hbm, sidx_v, didx_v, buf, acc_sh, sem_g, sem_s,
                   nbatch=epw // BATCH)

  @pl.when(c == 1)
  def _():
    _run_edge_pass(zb_hbm, sidx_v, didx_v, buf, acc_sh, sem_g, sem_s,
                   nbatch=epw // BATCH)

  plsc.subcore_barrier()
  _copy_out(acc_sh, o0, o1, c, s)


@functools.partial(
    pl.kernel,
    mesh=_mesh,
    compiler_params=_sc_params,
    out_type=(
        jax.ShapeDtypeStruct((N, D2), jnp.float32),
        jax.ShapeDtypeStruct((N, D2), jnp.float32),
    ),
    scratch_types=[
        pltpu.VMEM((EPW,), jnp.int32),
        pltpu.VMEM((NCHUNK, CH), jnp.int32),
        pltpu.VMEM((BATCH, D2), jnp.float32),
        pltpu.VMEM_SHARED((N, D2), jnp.float32),
        pltpu.SemaphoreType.DMA,
        pltpu.SemaphoreType.DMA,
    ],
)
def _agg16(z_hbm, src_hbm, dst3_hbm, zero_hbm, outa, outb,
           sidx_v, didx_v, buf, acc_sh, sem_g, sem_s):
  c, s, wid = _worker_prologue(dst3_hbm, didx_v)
  pltpu.sync_copy(src_hbm.at[pl.ds(pl.multiple_of(wid * EPW, 8), EPW)],
                  sidx_v)
  _zero_acc(zero_hbm, acc_sh, s)
  plsc.subcore_barrier()
  _run_edge_pass(z_hbm, sidx_v, didx_v, buf, acc_sh, sem_g, sem_s)
  plsc.subcore_barrier()
  _copy_out(acc_sh, outa, outb, c, s)


@functools.partial(
    pl.kernel,
    mesh=_mesh,
    compiler_params=_sc_params,
    out_type=(
        jax.ShapeDtypeStruct((N, D2), jnp.float32),
        jax.ShapeDtypeStruct((N, D2), jnp.float32),
    ),
    scratch_types=[
        pltpu.VMEM((NCHUNK, CH), jnp.int32),
        pltpu.VMEM((CH, D2), jnp.float32),
        pltpu.VMEM_SHARED((N, D2), jnp.float32),
        pltpu.SemaphoreType.DMA,
    ],
)
def _deg_kernel(dst3_hbm, ones_hbm, zero_hbm, outa, outb,
                didx_v, ones_v, acc_sh, sem):
  c, s, wid = _worker_prologue(dst3_hbm, didx_v)
  pltpu.sync_copy(ones_hbm, ones_v)
  _zero_acc(zero_hbm, acc_sh, s)
  plsc.subcore_barrier()

  def body(t, carry):
    handles = []
    for b in range(K):
      handles.append(
          pltpu.async_copy(ones_v, acc_sh.at[didx_v.at[t * K + b]], sem,
                           add=True))
    for h in handles:
      h.wait()
    return carry

  lax.fori_loop(0, NBATCH, body, 0)
  plsc.subcore_barrier()
  _copy_out(acc_sh, outa, outb, c, s)


BLK = 1000


def _z1_body(x_ref, w_ref, da_ref, db_ref, oa_ref, ob_ref):
  deg = da_ref[:, 0:1] + db_ref[:, 0:1] + 1.0
  dis = lax.rsqrt(deg)
  z = jnp.dot(x_ref[...], w_ref[...],
              preferred_element_type=jnp.float32) * dis
  oa_ref[...] = z[:, 0:DH]
  ob_ref[...] = z[:, DH:D1]


def _z2_body(s0_ref, s1_ref, za_ref, zb_ref,
             da_ref, db_ref, b1_ref, w2_ref, o_ref):
  deg = da_ref[:, 0:1] + db_ref[:, 0:1] + 1.0
  dis = lax.rsqrt(deg)
  hl = (s0_ref[...] + za_ref[...]) * dis + b1_ref[:, 0:DH]
  hh = (s1_ref[...] + zb_ref[...]) * dis + b1_ref[:, DH:D1]
  h = jnp.maximum(jnp.concatenate([hl, hh], axis=1), 0.0)
  o_ref[...] = jnp.dot(h, w2_ref[...], preferred_element_type=jnp.float32) * dis


def _out_body(sa_ref, sb_ref, z2_ref, da_ref, db_ref, b2_ref, o_ref):
  deg = da_ref[:, 0:1] + db_ref[:, 0:1] + 1.0
  dis = lax.rsqrt(deg)
  y = (sa_ref[...] + sb_ref[...] + z2_ref[...]) * dis
  o_ref[...] = y[:, 0:NCLS] + b2_ref[...]


def _row_spec(d):
  return pl.BlockSpec((BLK, d), lambda i: (i, 0))


def _full_spec(r, c):
  return pl.BlockSpec((r, c), lambda i: (0, 0))


def kernel(x, edge_index, W1, b1, W2, b2):
  src = edge_index[0].astype(jnp.int32)
  dst = edge_index[1].astype(jnp.int32)
  dst3 = dst.reshape(NW, NCHUNK, CH)
  dst16 = dst.reshape(NS, E // NS // CH, CH)
  ones_rows = jnp.ones((CH, D2), jnp.float32)
  zeros16 = jnp.zeros((N, D2), jnp.float32)
  zeros64 = jnp.zeros((N, DH), jnp.float32)
  W2p = jnp.pad(W2, ((0, 0), (0, D2 - NCLS)))
  b1r = b1.reshape(1, D1)
  b2r = b2.reshape(1, NCLS)

  dega, degb = _deg_kernel(dst3, ones_rows, zeros16)

  z1a, z1b = pl.pallas_call(
      _z1_body,
      grid=(N // BLK,),
      in_specs=[_row_spec(D1), _full_spec(D1, D1), _row_spec(D2),
                _row_spec(D2)],
      out_specs=(_row_spec(DH), _row_spec(DH)),
      out_shape=(jax.ShapeDtypeStruct((N, DH), jnp.float32),
                 jax.ShapeDtypeStruct((N, DH), jnp.float32)),
  )(x, W1, dega, degb)

  s0, s1 = _agg64split(z1a, z1b, src, dst16, zeros64)

  z2 = pl.pallas_call(
      _z2_body,
      grid=(N // BLK,),
      in_specs=[_row_spec(DH)] * 4 + [_row_spec(D2), _row_spec(D2),
                _full_spec(1, D1), _full_spec(D1, D2)],
      out_specs=_row_spec(D2),
      out_shape=jax.ShapeDtypeStruct((N, D2), jnp.float32),
  )(s0, s1, z1a, z1b, dega, degb, b1r, W2p)

  s2a, s2b = _agg16(z2, src, dst3, zeros16)

  out = pl.pallas_call(
      _out_body,
      grid=(N // BLK,),
      in_specs=[_row_spec(D2), _row_spec(D2), _row_spec(D2), _row_spec(D2),
                _row_spec(D2), _full_spec(1, NCLS)],
      out_specs=_row_spec(NCLS),
      out_shape=jax.ShapeDtypeStruct((N, NCLS), jnp.float32),
  )(s2a, s2b, z2, dega, degb, b2r)

  return out


# speedup vs baseline: 35.3547x; 1.1631x over previous
import functools

import jax
import jax.numpy as jnp
from jax import lax
from jax.experimental import pallas as pl
from jax.experimental.pallas import tpu as pltpu
from jax.experimental.pallas import tpu_sc as plsc

N = 10000
E = 320000
D1 = 128
DH = 64
D2 = 16
NCLS = 8

NC = 2
NS = 16
NW = NC * NS
CH = 80
NCHG = E // CH
K = 5
BATCH = K * CH
RPT = 624
TAIL = N - NS * RPT

_mesh = plsc.VectorSubcoreMesh(core_axis_name="c", subcore_axis_name="s")
_sc_params = pltpu.CompilerParams(use_tc_tiling_on_sc=False)


def _zero_acc(zero_hbm, acc_sh, s):
  r0 = pl.multiple_of(s * RPT, 8)
  pltpu.sync_copy(zero_hbm.at[pl.ds(r0, RPT)], acc_sh.at[pl.ds(r0, RPT)])

  @pl.when(s == 0)
  def _():
    pltpu.sync_copy(zero_hbm.at[pl.ds(NS * RPT, TAIL)],
                    acc_sh.at[pl.ds(NS * RPT, TAIL)])


def _copy_out(acc_sh, outa, outb, c, s):
  r0 = pl.multiple_of(s * RPT, 8)

  @pl.when(c == 0)
  def _():
    pltpu.sync_copy(acc_sh.at[pl.ds(r0, RPT)], outa.at[pl.ds(r0, RPT)])

    @pl.when(s == 0)
    def _():
      pltpu.sync_copy(acc_sh.at[pl.ds(NS * RPT, TAIL)],
                      outa.at[pl.ds(NS * RPT, TAIL)])

  @pl.when(c == 1)
  def _():
    pltpu.sync_copy(acc_sh.at[pl.ds(r0, RPT)], outb.at[pl.ds(r0, RPT)])

    @pl.when(s == 0)
    def _():
      pltpu.sync_copy(acc_sh.at[pl.ds(NS * RPT, TAIL)],
                      outb.at[pl.ds(NS * RPT, TAIL)])


def _run_edge_pass(z_hbm, src_hbm, dst2_hbm, base_chunk, nbatch,
                   sidx3, didx3, bufa, bufb, acc_sh,
                   sem_g, sem_sa, sem_sb, sem_st):
  def fire_stage(t):
    ck = base_chunk + t * K
    pltpu.async_copy(src_hbm.at[pl.ds(pl.multiple_of(ck * CH, 8), BATCH)],
                     sidx3.at[t % 3], sem_st)
    pltpu.async_copy(dst2_hbm.at[pl.ds(ck, K)], didx3.at[t % 3], sem_st)

  def drain_stage(t):
    ck = base_chunk + t * K
    pltpu.make_async_copy(
        src_hbm.at[pl.ds(pl.multiple_of(ck * CH, 8), BATCH)],
        sidx3.at[t % 3], sem_st).wait()
    pltpu.make_async_copy(dst2_hbm.at[pl.ds(ck, K)], didx3.at[t % 3],
                          sem_st).wait()

  def gather_descs(t, buf):
    return [pltpu.make_async_copy(
        z_hbm.at[sidx3.at[t % 3].at[pl.ds(b * CH, CH)]],
        buf.at[pl.ds(b * CH, CH)], sem_g) for b in range(K)]

  def scatter_descs(t, buf, sem):
    return [pltpu.make_async_copy(
        buf.at[pl.ds(b * CH, CH)],
        acc_sh.at[didx3.at[t % 3].at[b]], sem) for b in range(K)]

  def do_batch(t, buf_p, sem_p, buf_q, sem_q):
    for d in gather_descs(t, buf_p):
      d.wait()

    @pl.when(t > 0)
    def _():
      for d in scatter_descs(t - 1, buf_q, sem_q):
        d.wait()

    @pl.when(t + 1 < nbatch)
    def _():
      for d in gather_descs(t + 1, buf_q):
        d.start()

    @pl.when(t + 2 < nbatch)
    def _():
      fire_stage(t + 2)

    for b in range(K):
      pltpu.async_copy(buf_p.at[pl.ds(b * CH, CH)],
                       acc_sh.at[didx3.at[t % 3].at[b]], sem_p, add=True)

  fire_stage(0)
  drain_stage(0)
  if nbatch > 1:
    fire_stage(1)
  for d in gather_descs(0, bufa):
    d.start()

  def body(t, carry):
    @pl.when(t + 1 < nbatch)
    def _():
      drain_stage(t + 1)

    @pl.when(t % 2 == 0)
    def _():
      do_batch(t, bufa, sem_sa, bufb, sem_sb)

    @pl.when(t % 2 == 1)
    def _():
      do_batch(t, bufb, sem_sb, bufa, sem_sa)

    return carry

  lax.fori_loop(0, nbatch, body, 0)
  last = nbatch - 1
  lb, ls = (bufa, sem_sa) if last % 2 == 0 else (bufb, sem_sb)
  for d in scatter_descs(last, lb, ls):
    d.wait()


def _edge_scratch(d):
  return [
      pltpu.VMEM((3, BATCH), jnp.int32),
      pltpu.VMEM((3, K, CH), jnp.int32),
      pltpu.VMEM((BATCH, d), jnp.float32),
      pltpu.VMEM((BATCH, d), jnp.float32),
      pltpu.VMEM_SHARED((N, d), jnp.float32),
      pltpu.SemaphoreType.DMA,
      pltpu.SemaphoreType.DMA,
      pltpu.SemaphoreType.DMA,
      pltpu.SemaphoreType.DMA,
  ]


@functools.partial(
    pl.kernel,
    mesh=_mesh,
    compiler_params=_sc_params,
    out_type=(
        jax.ShapeDtypeStruct((N, DH), jnp.float32),
        jax.ShapeDtypeStruct((N, DH), jnp.float32),
    ),
    scratch_types=_edge_scratch(DH),
)
def _agg64split(za_hbm, zb_hbm, src_hbm, dst2_hbm, zero_hbm, o0, o1,
                sidx3, didx3, bufa, bufb, acc_sh,
                sem_g, sem_sa, sem_sb, sem_st):
  c = lax.axis_index("c")
  s = lax.axis_index("s")
  nbatch = E // NS // BATCH
  base = s * (NCHG // NS)
  _zero_acc(zero_hbm, acc_sh, s)
  plsc.subcore_barrier()

  @pl.when(c == 0)
  def _():
    _run_edge_pass(za_hbm, src_hbm, dst2_hbm, base, nbatch,
                   sidx3, didx3, bufa, bufb, acc_sh,
                   sem_g, sem_sa, sem_sb, sem_st)

  @pl.when(c == 1)
  def _():
    _run_edge_pass(zb_hbm, src_hbm, dst2_hbm, base, nbatch,
                   sidx3, didx3, bufa, bufb, acc_sh,
                   sem_g, sem_sa, sem_sb, sem_st)

  plsc.subcore_barrier()
  _copy_out(acc_sh, o0, o1, c, s)


@functools.partial(
    pl.kernel,
    mesh=_mesh,
    compiler_params=_sc_params,
    out_type=(
        jax.ShapeDtypeStruct((N, D2), jnp.float32),
        jax.ShapeDtypeStruct((N, D2), jnp.float32),
    ),
    scratch_types=_edge_scratch(D2),
)
def _agg16(z_hbm, src_hbm, dst2_hbm, zero_hbm, outa, outb,
           sidx3, didx3, bufa, bufb, acc_sh,
           sem_g, sem_sa, sem_sb, sem_st):
  c = lax.axis_index("c")
  s = lax.axis_index("s")
  wid = s * NC + c
  nbatch = E // NW // BATCH
  base = wid * (NCHG // NW)
  _zero_acc(zero_hbm, acc_sh, s)
  plsc.subcore_barrier()
  _run_edge_pass(z_hbm, src_hbm, dst2_hbm, base, nbatch,
                 sidx3, didx3, bufa, bufb, acc_sh,
                 sem_g, sem_sa, sem_sb, sem_st)
  plsc.subcore_barrier()
  _copy_out(acc_sh, outa, outb, c, s)


@functools.partial(
    pl.kernel,
    mesh=_mesh,
    compiler_params=_sc_params,
    out_type=(
        jax.ShapeDtypeStruct((N, D2), jnp.float32),
        jax.ShapeDtypeStruct((N, D2), jnp.float32),
    ),
    scratch_types=[
        pltpu.VMEM((E // NW // CH, CH), jnp.int32),
        pltpu.VMEM((CH, D2), jnp.float32),
        pltpu.VMEM_SHARED((N, D2), jnp.float32),
        pltpu.SemaphoreType.DMA,
    ],
)
def _deg_kernel(dst2_hbm, ones_hbm, zero_hbm, outa, outb,
                didx_v, ones_v, acc_sh, sem):
  c = lax.axis_index("c")
  s = lax.axis_index("s")
  wid = s * NC + c
  nchunk = E // NW // CH
  pltpu.sync_copy(dst2_hbm.at[pl.ds(wid * nchunk, nchunk)], didx_v)
  pltpu.sync_copy(ones_hbm, ones_v)
  _zero_acc(zero_hbm, acc_sh, s)
  plsc.subcore_barrier()

  def body(t, carry):
    handles = []
    for b in range(K):
      handles.append(
          pltpu.async_copy(ones_v, acc_sh.at[didx_v.at[t * K + b]], sem,
                           add=True))
    for h in handles:
      h.wait()
    return carry

  lax.fori_loop(0, nchunk // K, body, 0)
  plsc.subcore_barrier()
  _copy_out(acc_sh, outa, outb, c, s)


BLK = 1000


def _z1_body(x_ref, w_ref, da_ref, db_ref, oa_ref, ob_ref):
  deg = da_ref[:, 0:1] + db_ref[:, 0:1] + 1.0
  dis = lax.rsqrt(deg)
  z = jnp.dot(x_ref[...], w_ref[...],
              preferred_element_type=jnp.float32) * dis
  oa_ref[...] = z[:, 0:DH]
  ob_ref[...] = z[:, DH:D1]


def _z2_body(s0_ref, s1_ref, za_ref, zb_ref,
             da_ref, db_ref, b1_ref, w2_ref, o_ref):
  deg = da_ref[:, 0:1] + db_ref[:, 0:1] + 1.0
  dis = lax.rsqrt(deg)
  hl = (s0_ref[...] + za_ref[...]) * dis + b1_ref[:, 0:DH]
  hh = (s1_ref[...] + zb_ref[...]) * dis + b1_ref[:, DH:D1]
  h = jnp.maximum(jnp.concatenate([hl, hh], axis=1), 0.0)
  o_ref[...] = jnp.dot(h, w2_ref[...], preferred_element_type=jnp.float32) * dis


def _out_body(sa_ref, sb_ref, z2_ref, da_ref, db_ref, b2_ref, o_ref):
  deg = da_ref[:, 0:1] + db_ref[:, 0:1] + 1.0
  dis = lax.rsqrt(deg)
  y = (sa_ref[...] + sb_ref[...] + z2_ref[...]) * dis
  o_ref[...] = y[:, 0:NCLS] + b2_ref[...]


def _row_spec(d):
  return pl.BlockSpec((BLK, d), lambda i: (i, 0))


def _full_spec(r, c):
  return pl.BlockSpec((r, c), lambda i: (0, 0))


def kernel(x, edge_index, W1, b1, W2, b2):
  src = edge_index[0].astype(jnp.int32)
  dst = edge_index[1].astype(jnp.int32)
  dst2 = dst.reshape(NCHG, CH)
  ones_rows = jnp.ones((CH, D2), jnp.float32)
  zeros16 = jnp.zeros((N, D2), jnp.float32)
  zeros64 = jnp.zeros((N, DH), jnp.float32)
  W2p = jnp.pad(W2, ((0, 0), (0, D2 - NCLS)))
  b1r = b1.reshape(1, D1)
  b2r = b2.reshape(1, NCLS)

  dega, degb = _deg_kernel(dst2, ones_rows, zeros16)

  z1a, z1b = pl.pallas_call(
      _z1_body,
      grid=(N // BLK,),
      in_specs=[_row_spec(D1), _full_spec(D1, D1), _row_spec(D2),
                _row_spec(D2)],
      out_specs=(_row_spec(DH), _row_spec(DH)),
      out_shape=(jax.ShapeDtypeStruct((N, DH), jnp.float32),
                 jax.ShapeDtypeStruct((N, DH), jnp.float32)),
  )(x, W1, dega, degb)

  s0, s1 = _agg64split(z1a, z1b, src, dst2, zeros64)

  z2 = pl.pallas_call(
      _z2_body,
      grid=(N // BLK,),
      in_specs=[_row_spec(DH)] * 4 + [_row_spec(D2), _row_spec(D2),
                _full_spec(1, D1), _full_spec(D1, D2)],
      out_specs=_row_spec(D2),
      out_shape=jax.ShapeDtypeStruct((N, D2), jnp.float32),
  )(s0, s1, z1a, z1b, dega, degb, b1r, W2p)

  s2a, s2b = _agg16(z2, src, dst2, zeros16)

  out = pl.pallas_call(
      _out_body,
      grid=(N // BLK,),
      in_specs=[_row_spec(D2), _row_spec(D2), _row_spec(D2), _row_spec(D2),
                _row_spec(D2), _full_spec(1, NCLS)],
      out_specs=_row_spec(NCLS),
      out_shape=jax.ShapeDtypeStruct((N, NCLS), jnp.float32),
  )(s2a, s2b, z2, dega, degb, b2r)

  return out
